# baseline (speedup 1.0000x reference)
import numpy as np
import jax
import jax.numpy as jnp
from jax.experimental import pallas as pl
from jax.experimental.pallas import tpu as pltpu

D_MODEL = 768
NUM_HEADS = 12
U = 64
ROW_BLOCK = 512
T_SEQ = 2048
KEY_CHUNK = 512
SLICES = 3
BH_TOTAL = 24

_T0 = [[0] * NUM_HEADS for _ in range(SLICES)]
_LEN = [[0] * NUM_HEADS for _ in range(SLICES)]
_BASE = [[0] * NUM_HEADS for _ in range(SLICES)]
for _r in range(SLICES):
    _acc = 0
    for _h in range(NUM_HEADS):
        _t0 = -((-(T_SEQ * _r - _h)) // NUM_HEADS)
        _t1 = -((-(T_SEQ * (_r + 1) - _h)) // NUM_HEADS)
        _T0[_r][_h] = _t0
        _LEN[_r][_h] = _t1 - _t0
        _BASE[_r][_h] = _acc
        _acc += _t1 - _t0
    assert _acc == T_SEQ

_SMAP = np.zeros((SLICES, T_SEQ), dtype=np.int32)
for _r in range(SLICES):
    for _h in range(NUM_HEADS):
        _i = np.arange(_LEN[_r][_h])
        _SMAP[_r, _BASE[_r][_h] + _i] = (
            NUM_HEADS * (_T0[_r][_h] + _i) + _h - T_SEQ * _r)
for _r in range(SLICES):
    assert np.array_equal(np.sort(_SMAP[_r]), np.arange(T_SEQ))


def _stack_slice(mat, r):
    pieces = []
    for h in range(NUM_HEADS):
        t0 = _T0[r][h]
        ln = _LEN[r][h]
        pieces.append(mat[t0:t0 + ln, 64 * h:64 * (h + 1)])
    return jnp.concatenate(pieces, axis=0)


def _unstack_block(slices_):
    cols = []
    for h in range(NUM_HEADS):
        segs = []
        for r in range(SLICES):
            b = _BASE[r][h]
            ln = _LEN[r][h]
            segs.append(slices_[r][b:b + ln, :])
        cols.append(jnp.concatenate(segs, axis=0))
    return jnp.concatenate(cols, axis=1)


def _stage1_kernel(smap24_ref, x_ref, wq_ref, wk_ref, wv_ref, bq_ref,
                   bk_ref, bv_ref, qb_ref, kb_ref, vb_ref, idx_ref, m_scr):
    g = pl.program_id(0)
    xb = x_ref[...]
    Qg = jnp.dot(xb, wq_ref[...], preferred_element_type=jnp.float32) + bq_ref[...]
    Kg = jnp.dot(xb, wk_ref[...], preferred_element_type=jnp.float32) + bk_ref[...]
    Vg = jnp.dot(xb, wv_ref[...], preferred_element_type=jnp.float32) + bv_ref[...]
    for r in range(SLICES):
        qs = _stack_slice(Qg, r)
        ks = _stack_slice(Kg, r)
        vs = _stack_slice(Vg, r)
        qb_ref[r] = qs.astype(jnp.bfloat16)
        kb_ref[r] = ks.astype(jnp.bfloat16)
        vb_ref[r] = vs.astype(jnp.bfloat16)
        nq = qs * jax.lax.rsqrt(jnp.maximum(jnp.sum(qs * qs, axis=-1, keepdims=True), 1e-12))
        nk = ks * jax.lax.rsqrt(jnp.maximum(jnp.sum(ks * ks, axis=-1, keepdims=True), 1e-12))
        m = jnp.full((1, T_SEQ), -jnp.inf, dtype=jnp.float32)
        for c in range(T_SEQ // KEY_CHUNK):
            kc = nk[c * KEY_CHUNK:(c + 1) * KEY_CHUNK]
            st = jax.lax.dot_general(kc, nq, (((1,), (1,)), ((), ())),
                                     preferred_element_type=jnp.float32)
            m = jnp.maximum(m, jnp.max(st, axis=0, keepdims=True))
        m_scr[SLICES * g + r] = m

    @pl.when(g == pl.num_programs(0) - 1)
    def _topk():
        smap24 = smap24_ref[...]
        v0 = m_scr[...].reshape(BH_TOTAL, T_SEQ)
        rank_iota = jax.lax.broadcasted_iota(jnp.int32, (BH_TOTAL, U), 1)
        acc0 = jnp.zeros((BH_TOTAL, U), dtype=jnp.int32)

        def body(r, carry):
            v, acc = carry
            mx = jnp.max(v, axis=1, keepdims=True)
            oh = v == mx
            idx = jnp.max(jnp.where(oh, smap24, -1), axis=1, keepdims=True)
            acc = jnp.where(rank_iota == r, idx, acc)
            return jnp.where(smap24 == idx, -jnp.inf, v), acc

        _, acc = jax.lax.fori_loop(0, U, body, (v0, acc0))
        idx_ref[...] = acc


def _stage2_kernel(smap3_ref, smapT_ref, idx_in_ref, qb_ref, kb_ref,
                   vb_ref, wo_ref, bo_ref, y_ref):
    g = pl.program_id(0)
    idx_all = idx_in_ref[...]

    bh_iota = jax.lax.broadcasted_iota(jnp.int32, (BH_TOTAL, U), 0)
    outs = []
    for r in range(SLICES):
        j = SLICES * g + r
        Qb = qb_ref[r]
        Kb = kb_ref[r]
        Vb = vb_ref[r]
        idx_row = jnp.max(jnp.where(bh_iota == j, idx_all, -1), axis=0,
                          keepdims=True)
        idx_col = jnp.reshape(idx_row, (U, 1))
        smap_row = smap3_ref[r:r + 1, :]
        smap_col = smapT_ref[...][:, r:r + 1]
        P = (idx_col == smap_row).astype(jnp.bfloat16)
        Pt = (smap_col == idx_row).astype(jnp.bfloat16)

        q_sel = jnp.dot(P, Qb, preferred_element_type=jnp.float32)
        s = jax.lax.dot_general(q_sel.astype(jnp.bfloat16), Kb,
                                (((1,), (1,)), ((), ())),
                                preferred_element_type=jnp.float32) * 0.125
        s = s - jnp.max(s, axis=-1, keepdims=True)
        e = jnp.exp(s)
        a = (e / jnp.sum(e, axis=-1, keepdims=True)).astype(jnp.bfloat16)
        out = jnp.dot(a, Vb, preferred_element_type=jnp.float32)
        outs.append(jnp.dot(Pt, out.astype(jnp.bfloat16),
                            preferred_element_type=jnp.float32
                            ).astype(jnp.bfloat16))

    blk = _unstack_block(outs)
    y_ref[...] = jnp.dot(blk, wo_ref[...],
                         preferred_element_type=jnp.float32) + bo_ref[...]


def kernel(x, Wq, bq, Wk, bk, Wv, bv, Wo, bo):
    B, T, d_model = x.shape
    H = NUM_HEADS
    d_k = d_model // H
    BT = B * T
    BH = B * H

    x2 = x.reshape(BT, d_model)
    bq2 = bq.reshape(1, d_model)
    bk2 = bk.reshape(1, d_model)
    bv2 = bv.reshape(1, d_model)
    bo2 = bo.reshape(1, d_model)

    n_row = BT // ROW_BLOCK
    row_spec = pl.BlockSpec((ROW_BLOCK, d_model), lambda i: (i, 0))
    w_spec = pl.BlockSpec((d_model, d_model), lambda i: (0, 0))
    b_spec = pl.BlockSpec((1, d_model), lambda i: (0, 0))
    slc_spec = pl.BlockSpec((SLICES, T, d_k), lambda i: (i, 0, 0))
    m_spec = pl.BlockSpec((SLICES, 1, T), lambda i: (i, 0, 0))

    smap24_arr = jnp.asarray(np.tile(_SMAP, (BH // SLICES, 1)))
    smap3_arr = jnp.asarray(_SMAP.copy())
    smapT_arr = jnp.asarray(_SMAP.T.copy())
    smap24_spec = pl.BlockSpec((BH, T), lambda i: (0, 0))

    qb, kb, vb, top_idx = pl.pallas_call(
        _stage1_kernel,
        grid=(n_row,),
        in_specs=[smap24_spec, row_spec, w_spec, w_spec, w_spec, b_spec,
                  b_spec, b_spec],
        out_specs=[slc_spec, slc_spec, slc_spec,
                   pl.BlockSpec((BH, U), lambda i: (0, 0))],
        out_shape=[jax.ShapeDtypeStruct((BH, T, d_k), jnp.bfloat16),
                   jax.ShapeDtypeStruct((BH, T, d_k), jnp.bfloat16),
                   jax.ShapeDtypeStruct((BH, T, d_k), jnp.bfloat16),
                   jax.ShapeDtypeStruct((BH, U), jnp.int32)],
        scratch_shapes=[pltpu.VMEM((BH, 1, T), jnp.float32)],
    )(smap24_arr, x2, Wq, Wk, Wv, bq2, bk2, bv2)

    smap3_spec = pl.BlockSpec((SLICES, T), lambda i: (0, 0))
    smapT_spec = pl.BlockSpec((T, SLICES), lambda i: (0, 0))
    idx_spec = pl.BlockSpec((BH, U), lambda i: (0, 0))
    y = pl.pallas_call(
        _stage2_kernel,
        grid=(n_row,),
        in_specs=[smap3_spec, smapT_spec, idx_spec, slc_spec, slc_spec,
                  slc_spec, w_spec, b_spec],
        out_specs=row_spec,
        out_shape=jax.ShapeDtypeStruct((BT, d_model), jnp.float32),
    )(smap3_arr, smapT_arr, top_idx, qb, kb, vb, Wo.astype(jnp.bfloat16), bo2)

    return y.reshape(B, T, d_model)


# speedup vs baseline: 1.0068x; 1.0068x over previous
import numpy as np
import jax
import jax.numpy as jnp
from jax.experimental import pallas as pl
from jax.experimental.pallas import tpu as pltpu

D_MODEL = 768
NUM_HEADS = 12
U = 64
ROW_BLOCK = 512
T_SEQ = 2048
KEY_CHUNK = 512
SLICES = 3
BH_TOTAL = 24

_T0 = [[0] * NUM_HEADS for _ in range(SLICES)]
_LEN = [[0] * NUM_HEADS for _ in range(SLICES)]
_BASE = [[0] * NUM_HEADS for _ in range(SLICES)]
for _r in range(SLICES):
    _acc = 0
    for _h in range(NUM_HEADS):
        _t0 = -((-(T_SEQ * _r - _h)) // NUM_HEADS)
        _t1 = -((-(T_SEQ * (_r + 1) - _h)) // NUM_HEADS)
        _T0[_r][_h] = _t0
        _LEN[_r][_h] = _t1 - _t0
        _BASE[_r][_h] = _acc
        _acc += _t1 - _t0
    assert _acc == T_SEQ

_SMAP = np.zeros((SLICES, T_SEQ), dtype=np.int32)
for _r in range(SLICES):
    for _h in range(NUM_HEADS):
        _i = np.arange(_LEN[_r][_h])
        _SMAP[_r, _BASE[_r][_h] + _i] = (
            NUM_HEADS * (_T0[_r][_h] + _i) + _h - T_SEQ * _r)
for _r in range(SLICES):
    assert np.array_equal(np.sort(_SMAP[_r]), np.arange(T_SEQ))


def _stack_slice(mat, r):
    pieces = []
    for h in range(NUM_HEADS):
        t0 = _T0[r][h]
        ln = _LEN[r][h]
        pieces.append(mat[t0:t0 + ln, 64 * h:64 * (h + 1)])
    return jnp.concatenate(pieces, axis=0)


def _unstack_block(slices_):
    cols = []
    for h in range(NUM_HEADS):
        segs = []
        for r in range(SLICES):
            b = _BASE[r][h]
            ln = _LEN[r][h]
            segs.append(slices_[r][b:b + ln, :])
        cols.append(jnp.concatenate(segs, axis=0))
    return jnp.concatenate(cols, axis=1)


def _stage1_kernel(x_ref, wq_ref, wk_ref, wv_ref, bq_ref, bk_ref, bv_ref,
                   qb_ref, kb_ref, vb_ref, m_ref):
    xb = x_ref[...]
    Qg = jnp.dot(xb, wq_ref[...], preferred_element_type=jnp.float32) + bq_ref[...]
    Kg = jnp.dot(xb, wk_ref[...], preferred_element_type=jnp.float32) + bk_ref[...]
    Vg = jnp.dot(xb, wv_ref[...], preferred_element_type=jnp.float32) + bv_ref[...]
    for r in range(SLICES):
        qs = _stack_slice(Qg, r)
        ks = _stack_slice(Kg, r)
        vs = _stack_slice(Vg, r)
        qb_ref[r] = qs.astype(jnp.bfloat16)
        kb_ref[r] = ks.astype(jnp.bfloat16)
        vb_ref[r] = vs.astype(jnp.bfloat16)
        nq = qs * jax.lax.rsqrt(jnp.maximum(jnp.sum(qs * qs, axis=-1, keepdims=True), 1e-12))
        nk = ks * jax.lax.rsqrt(jnp.maximum(jnp.sum(ks * ks, axis=-1, keepdims=True), 1e-12))
        m = jnp.full((1, T_SEQ), -jnp.inf, dtype=jnp.float32)
        for c in range(T_SEQ // KEY_CHUNK):
            kc = nk[c * KEY_CHUNK:(c + 1) * KEY_CHUNK]
            st = jax.lax.dot_general(kc, nq, (((1,), (1,)), ((), ())),
                                     preferred_element_type=jnp.float32)
            m = jnp.maximum(m, jnp.max(st, axis=0, keepdims=True))
        m_ref[r] = m


def _stage2_kernel(smap24_ref, smapT_ref, m_ref, qb_ref, kb_ref, vb_ref,
                   wo_ref, bo_ref, y_ref, idx_scr):
    g = pl.program_id(0)
    smap24 = smap24_ref[...]

    @pl.when(g == 0)
    def _topk():
        v0 = m_ref[...].reshape(BH_TOTAL, T_SEQ)
        rank_iota = jax.lax.broadcasted_iota(jnp.int32, (BH_TOTAL, U), 1)
        acc0 = jnp.zeros((BH_TOTAL, U), dtype=jnp.int32)

        def body(r, carry):
            v, acc = carry
            mx = jnp.max(v, axis=1, keepdims=True)
            oh = v == mx
            idx = jnp.max(jnp.where(oh, smap24, -1), axis=1, keepdims=True)
            acc = jnp.where(rank_iota == r, idx, acc)
            return jnp.where(smap24 == idx, -jnp.inf, v), acc

        _, acc = jax.lax.fori_loop(0, U, body, (v0, acc0))
        idx_scr[...] = acc

    bh_iota = jax.lax.broadcasted_iota(jnp.int32, (BH_TOTAL, U), 0)
    outs = []
    for r in range(SLICES):
        j = SLICES * g + r
        Qb = qb_ref[r]
        Kb = kb_ref[r]
        Vb = vb_ref[r]
        idx_row = jnp.max(jnp.where(bh_iota == j, idx_scr[...], -1), axis=0,
                          keepdims=True)
        idx_col = jnp.reshape(idx_row, (U, 1))
        smap_row = smap24[r:r + 1, :]
        smap_col = smapT_ref[...][:, r:r + 1]
        P = (idx_col == smap_row).astype(jnp.bfloat16)
        Pt = (smap_col == idx_row).astype(jnp.bfloat16)

        q_sel = jnp.dot(P, Qb, preferred_element_type=jnp.float32)
        s = jax.lax.dot_general(q_sel.astype(jnp.bfloat16), Kb,
                                (((1,), (1,)), ((), ())),
                                preferred_element_type=jnp.float32) * 0.125
        s = s - jnp.max(s, axis=-1, keepdims=True)
        e = jnp.exp(s)
        a = (e / jnp.sum(e, axis=-1, keepdims=True)).astype(jnp.bfloat16)
        out = jnp.dot(a, Vb, preferred_element_type=jnp.float32)
        outs.append(jnp.dot(Pt, out.astype(jnp.bfloat16),
                            preferred_element_type=jnp.float32
                            ).astype(jnp.bfloat16))

    blk = _unstack_block(outs)
    y_ref[...] = jnp.dot(blk, wo_ref[...],
                         preferred_element_type=jnp.float32) + bo_ref[...]


def kernel(x, Wq, bq, Wk, bk, Wv, bv, Wo, bo):
    B, T, d_model = x.shape
    H = NUM_HEADS
    d_k = d_model // H
    BT = B * T
    BH = B * H

    x2 = x.reshape(BT, d_model)
    bq2 = bq.reshape(1, d_model)
    bk2 = bk.reshape(1, d_model)
    bv2 = bv.reshape(1, d_model)
    bo2 = bo.reshape(1, d_model)

    n_row = BT // ROW_BLOCK
    row_spec = pl.BlockSpec((ROW_BLOCK, d_model), lambda i: (i, 0))
    w_spec = pl.BlockSpec((d_model, d_model), lambda i: (0, 0))
    b_spec = pl.BlockSpec((1, d_model), lambda i: (0, 0))
    slc_spec = pl.BlockSpec((SLICES, T, d_k), lambda i: (i, 0, 0))
    m_spec = pl.BlockSpec((SLICES, 1, T), lambda i: (i, 0, 0))

    qb, kb, vb, m = pl.pallas_call(
        _stage1_kernel,
        grid=(n_row,),
        in_specs=[row_spec, w_spec, w_spec, w_spec, b_spec, b_spec, b_spec],
        out_specs=[slc_spec, slc_spec, slc_spec, m_spec],
        out_shape=[jax.ShapeDtypeStruct((BH, T, d_k), jnp.bfloat16),
                   jax.ShapeDtypeStruct((BH, T, d_k), jnp.bfloat16),
                   jax.ShapeDtypeStruct((BH, T, d_k), jnp.bfloat16),
                   jax.ShapeDtypeStruct((BH, 1, T), jnp.float32)],
        compiler_params=pltpu.CompilerParams(dimension_semantics=("parallel",)),
    )(x2, Wq, Wk, Wv, bq2, bk2, bv2)

    smap24_arr = jnp.asarray(np.tile(_SMAP, (BH // SLICES, 1)))
    smapT_arr = jnp.asarray(_SMAP.T.copy())

    m_full = pl.BlockSpec((BH, 1, T), lambda i: (0, 0, 0))
    smap24_spec = pl.BlockSpec((BH, T), lambda i: (0, 0))
    smapT_spec = pl.BlockSpec((T, SLICES), lambda i: (0, 0))
    y = pl.pallas_call(
        _stage2_kernel,
        grid=(n_row,),
        in_specs=[smap24_spec, smapT_spec, m_full, slc_spec, slc_spec,
                  slc_spec, w_spec, b_spec],
        out_specs=row_spec,
        out_shape=jax.ShapeDtypeStruct((BT, d_model), jnp.float32),
        scratch_shapes=[pltpu.VMEM((BH, U), jnp.int32)],
    )(smap24_arr, smapT_arr, m, qb, kb, vb, Wo.astype(jnp.bfloat16), bo2)

    return y.reshape(B, T, d_model)


# speedup vs baseline: 1.0248x; 1.0178x over previous
import numpy as np
import jax
import jax.numpy as jnp
from jax.experimental import pallas as pl
from jax.experimental.pallas import tpu as pltpu

D_MODEL = 768
NUM_HEADS = 12
U = 64
ROW_BLOCK = 512
T_SEQ = 2048
KEY_CHUNK = 512
SLICES = 3
BH_TOTAL = 24

_T0 = [[0] * NUM_HEADS for _ in range(SLICES)]
_LEN = [[0] * NUM_HEADS for _ in range(SLICES)]
_BASE = [[0] * NUM_HEADS for _ in range(SLICES)]
for _r in range(SLICES):
    _acc = 0
    for _h in range(NUM_HEADS):
        _t0 = -((-(T_SEQ * _r - _h)) // NUM_HEADS)
        _t1 = -((-(T_SEQ * (_r + 1) - _h)) // NUM_HEADS)
        _T0[_r][_h] = _t0
        _LEN[_r][_h] = _t1 - _t0
        _BASE[_r][_h] = _acc
        _acc += _t1 - _t0
    assert _acc == T_SEQ

_SMAP = np.zeros((SLICES, T_SEQ), dtype=np.int32)
for _r in range(SLICES):
    for _h in range(NUM_HEADS):
        _i = np.arange(_LEN[_r][_h])
        _SMAP[_r, _BASE[_r][_h] + _i] = (
            NUM_HEADS * (_T0[_r][_h] + _i) + _h - T_SEQ * _r)
for _r in range(SLICES):
    assert np.array_equal(np.sort(_SMAP[_r]), np.arange(T_SEQ))


def _stack_slice(mat, r):
    pieces = []
    for h in range(NUM_HEADS):
        t0 = _T0[r][h]
        ln = _LEN[r][h]
        pieces.append(mat[t0:t0 + ln, 64 * h:64 * (h + 1)])
    return jnp.concatenate(pieces, axis=0)


def _unstack_block(slices_):
    cols = []
    for h in range(NUM_HEADS):
        segs = []
        for r in range(SLICES):
            b = _BASE[r][h]
            ln = _LEN[r][h]
            segs.append(slices_[r][b:b + ln, :])
        cols.append(jnp.concatenate(segs, axis=0))
    return jnp.concatenate(cols, axis=1)


def _stage1_kernel(x_ref, wq_ref, wk_ref, wv_ref, bq_ref, bk_ref, bv_ref,
                   qb_ref, kb_ref, vb_ref, m_ref):
    xb = x_ref[...]
    Qg = jnp.dot(xb, wq_ref[...], preferred_element_type=jnp.float32) + bq_ref[...]
    Kg = jnp.dot(xb, wk_ref[...], preferred_element_type=jnp.float32) + bk_ref[...]
    Vg = jnp.dot(xb, wv_ref[...], preferred_element_type=jnp.float32) + bv_ref[...]
    for r in range(SLICES):
        qs = _stack_slice(Qg, r)
        ks = _stack_slice(Kg, r)
        vs = _stack_slice(Vg, r)
        qb_ref[r] = qs.astype(jnp.bfloat16)
        kb_ref[r] = ks.astype(jnp.bfloat16)
        vb_ref[r] = vs.astype(jnp.bfloat16)
        nq = qs * jax.lax.rsqrt(jnp.maximum(jnp.sum(qs * qs, axis=-1, keepdims=True), 1e-12))
        nk = ks * jax.lax.rsqrt(jnp.maximum(jnp.sum(ks * ks, axis=-1, keepdims=True), 1e-12))
        m = jnp.full((1, T_SEQ), -jnp.inf, dtype=jnp.float32)
        for c in range(T_SEQ // KEY_CHUNK):
            kc = nk[c * KEY_CHUNK:(c + 1) * KEY_CHUNK]
            st = jax.lax.dot_general(kc, nq, (((1,), (1,)), ((), ())),
                                     preferred_element_type=jnp.float32)
            m = jnp.maximum(m, jnp.max(st, axis=0, keepdims=True))
        m_ref[r] = m


def _stage2_kernel(smap24_ref, smapT_ref, m_ref, qb_ref, kb_ref, vb_ref,
                   wo_ref, bo_ref, y_ref, idx_scr):
    g = pl.program_id(0)
    smap24 = smap24_ref[...]

    @pl.when(g == 0)
    def _topk():
        v0 = m_ref[...].reshape(BH_TOTAL, T_SEQ)
        rank_iota = jax.lax.broadcasted_iota(jnp.int32, (BH_TOTAL, U), 1)
        acc0 = jnp.zeros((BH_TOTAL, U), dtype=jnp.int32)

        def body(r, carry):
            v, acc = carry
            mx = jnp.max(v, axis=1, keepdims=True)
            oh = v == mx
            idx = jnp.max(jnp.where(oh, smap24, -1), axis=1, keepdims=True)
            acc = jnp.where(rank_iota == r, idx, acc)
            return jnp.where(smap24 == idx, -jnp.inf, v), acc

        _, acc = jax.lax.fori_loop(0, U, body, (v0, acc0))
        idx_scr[...] = acc

    bh_iota = jax.lax.broadcasted_iota(jnp.int32, (BH_TOTAL, U), 0)
    outs = []
    for r in range(SLICES):
        j = SLICES * g + r
        Qb = qb_ref[r]
        Kb = kb_ref[r]
        Vb = vb_ref[r]
        idx_row = jnp.max(jnp.where(bh_iota == j, idx_scr[...], -1), axis=0,
                          keepdims=True)
        idx_col = jnp.reshape(idx_row, (U, 1))
        smap_row = smap24[r:r + 1, :]
        smap_col = smapT_ref[...][:, r:r + 1]
        P = (idx_col == smap_row).astype(jnp.bfloat16)
        Pt = (smap_col == idx_row).astype(jnp.bfloat16)

        q_sel = jnp.dot(P, Qb, preferred_element_type=jnp.float32)
        s = jax.lax.dot_general(q_sel.astype(jnp.bfloat16), Kb,
                                (((1,), (1,)), ((), ())),
                                preferred_element_type=jnp.float32) * 0.125
        s = s - jnp.max(s, axis=-1, keepdims=True)
        e = jnp.exp(s)
        a = (e / jnp.sum(e, axis=-1, keepdims=True)).astype(jnp.bfloat16)
        out = jnp.dot(a, Vb, preferred_element_type=jnp.float32)
        outs.append(jnp.dot(Pt, out.astype(jnp.bfloat16),
                            preferred_element_type=jnp.float32))

    blk = _unstack_block(outs)
    y_ref[...] = jnp.dot(blk, wo_ref[...],
                         preferred_element_type=jnp.float32) + bo_ref[...]


def kernel(x, Wq, bq, Wk, bk, Wv, bv, Wo, bo):
    B, T, d_model = x.shape
    H = NUM_HEADS
    d_k = d_model // H
    BT = B * T
    BH = B * H

    x2 = x.reshape(BT, d_model)
    bq2 = bq.reshape(1, d_model)
    bk2 = bk.reshape(1, d_model)
    bv2 = bv.reshape(1, d_model)
    bo2 = bo.reshape(1, d_model)

    n_row = BT // ROW_BLOCK
    row_spec = pl.BlockSpec((ROW_BLOCK, d_model), lambda i: (i, 0))
    w_spec = pl.BlockSpec((d_model, d_model), lambda i: (0, 0))
    b_spec = pl.BlockSpec((1, d_model), lambda i: (0, 0))
    slc_spec = pl.BlockSpec((SLICES, T, d_k), lambda i: (i, 0, 0))
    m_spec = pl.BlockSpec((SLICES, 1, T), lambda i: (i, 0, 0))

    qb, kb, vb, m = pl.pallas_call(
        _stage1_kernel,
        grid=(n_row,),
        in_specs=[row_spec, w_spec, w_spec, w_spec, b_spec, b_spec, b_spec],
        out_specs=[slc_spec, slc_spec, slc_spec, m_spec],
        out_shape=[jax.ShapeDtypeStruct((BH, T, d_k), jnp.bfloat16),
                   jax.ShapeDtypeStruct((BH, T, d_k), jnp.bfloat16),
                   jax.ShapeDtypeStruct((BH, T, d_k), jnp.bfloat16),
                   jax.ShapeDtypeStruct((BH, 1, T), jnp.float32)],
        compiler_params=pltpu.CompilerParams(dimension_semantics=("parallel",)),
    )(x2, Wq, Wk, Wv, bq2, bk2, bv2)

    smap24_arr = jnp.asarray(np.tile(_SMAP, (BH // SLICES, 1)))
    smapT_arr = jnp.asarray(_SMAP.T.copy())

    m_full = pl.BlockSpec((BH, 1, T), lambda i: (0, 0, 0))
    smap24_spec = pl.BlockSpec((BH, T), lambda i: (0, 0))
    smapT_spec = pl.BlockSpec((T, SLICES), lambda i: (0, 0))
    y = pl.pallas_call(
        _stage2_kernel,
        grid=(n_row,),
        in_specs=[smap24_spec, smapT_spec, m_full, slc_spec, slc_spec,
                  slc_spec, w_spec, b_spec],
        out_specs=row_spec,
        out_shape=jax.ShapeDtypeStruct((BT, d_model), jnp.float32),
        scratch_shapes=[pltpu.VMEM((BH, U), jnp.int32)],
    )(smap24_arr, smapT_arr, m, qb, kb, vb, Wo, bo2)

    return y.reshape(B, T, d_model)
